# baseline (device time: 16905 ns/iter reference)
import jax
import jax.numpy as jnp
from jax import lax
from jax.experimental import pallas as pl
from jax.experimental.pallas import tpu as pltpu

M = 1024
N = 1024
HALF = 512
K = 4
CW = HALF // K


def kernel(x):
    def body(
        x_hbm,
        out_hbm,
        vbuf_p,
        vbuf_m,
        send_y,
        recv_y,
        send_x,
        recv_x,
        hp_sem,
        hm_sem,
        sy_send,
        sy_recv,
        sx_send,
        sx_recv,
        so_sem,
    ):
        my_x = lax.axis_index("x")
        my_y = lax.axis_index("y")
        peer_x = 1 - my_x
        peer_y = 1 - my_y
        r0 = my_x * HALF
        pr0 = peer_x * HALF
        my_base = my_y * HALF
        peer_base = peer_y * HALF

        hbm_p = []
        hbm_m = []
        for c in range(K):
            col = c * CW
            cp = pltpu.make_async_copy(
                x_hbm.at[0, pl.ds(r0, HALF), pl.ds(peer_base + col, CW)],
                vbuf_p.at[c],
                hp_sem.at[c],
            )
            cp.start()
            hbm_p.append(cp)
        for c in range(K):
            col = c * CW
            cm = pltpu.make_async_copy(
                x_hbm.at[0, pl.ds(r0, HALF), pl.ds(my_base + col, CW)],
                vbuf_m.at[c],
                hm_sem.at[c],
            )
            cm.start()
            hbm_m.append(cm)

        barrier = pltpu.get_barrier_semaphore()
        for dev in ((my_x, peer_y), (peer_x, my_y)):
            pl.semaphore_signal(
                barrier, inc=1, device_id=dev,
                device_id_type=pl.DeviceIdType.MESH,
            )
        pl.semaphore_wait(barrier, 2)

        rdmas_y = []
        for c in range(K):
            hbm_p[c].wait()
            send_y[c] = vbuf_p[c].astype(jnp.bfloat16)
            rdma = pltpu.make_async_remote_copy(
                src_ref=send_y.at[c],
                dst_ref=recv_y.at[c],
                send_sem=sy_send.at[c],
                recv_sem=sy_recv.at[c],
                device_id=(my_x, peer_y),
                device_id_type=pl.DeviceIdType.MESH,
            )
            rdma.start()
            rdmas_y.append(rdma)

        rdmas_x = []
        out_copies = []
        for c in range(K):
            col = c * CW
            rdmas_y[c].wait_recv()
            hbm_m[c].wait()
            send_x[c] = vbuf_m[c].astype(jnp.bfloat16) + recv_y[c]
            rdma = pltpu.make_async_remote_copy(
                src_ref=send_x.at[c],
                dst_ref=recv_x.at[c],
                send_sem=sx_send.at[c],
                recv_sem=sx_recv.at[c],
                device_id=(peer_x, my_y),
                device_id_type=pl.DeviceIdType.MESH,
            )
            rdma.start()
            rdmas_x.append(rdma)

            cp = pltpu.make_async_copy(
                send_x.at[c],
                out_hbm.at[pl.ds(r0, HALF), pl.ds(col, CW)],
                so_sem.at[c],
            )
            cp.start()
            out_copies.append(cp)

        for c in range(K):
            col = c * CW
            rdmas_x[c].wait_recv()
            cp = pltpu.make_async_copy(
                recv_x.at[c],
                out_hbm.at[pl.ds(pr0, HALF), pl.ds(col, CW)],
                so_sem.at[K + c],
            )
            cp.start()
            out_copies.append(cp)

        for cp in out_copies:
            cp.wait()
        for c in range(K):
            rdmas_y[c].wait_send()
            rdmas_x[c].wait_send()

    return pl.pallas_call(
        body,
        out_shape=jax.ShapeDtypeStruct((M, HALF), jnp.bfloat16),
        in_specs=[pl.BlockSpec(memory_space=pl.ANY)],
        out_specs=pl.BlockSpec(memory_space=pl.ANY),
        scratch_shapes=[
            pltpu.VMEM((K, HALF, CW), jnp.float32),
            pltpu.VMEM((K, HALF, CW), jnp.float32),
            pltpu.VMEM((K, HALF, CW), jnp.bfloat16),
            pltpu.VMEM((K, HALF, CW), jnp.bfloat16),
            pltpu.VMEM((K, HALF, CW), jnp.bfloat16),
            pltpu.VMEM((K, HALF, CW), jnp.bfloat16),
            pltpu.SemaphoreType.DMA((K,)),
            pltpu.SemaphoreType.DMA((K,)),
            pltpu.SemaphoreType.DMA((K,)),
            pltpu.SemaphoreType.DMA((K,)),
            pltpu.SemaphoreType.DMA((K,)),
            pltpu.SemaphoreType.DMA((K,)),
            pltpu.SemaphoreType.DMA((2 * K,)),
        ],
        compiler_params=pltpu.CompilerParams(collective_id=0),
    )(x)


# device time: 16894 ns/iter; 1.0007x vs baseline; 1.0007x over previous
import jax
import jax.numpy as jnp
from jax import lax
from jax.experimental import pallas as pl
from jax.experimental.pallas import tpu as pltpu

M = 1024
N = 1024
HALF = 512
K = 4
CW = HALF // K


def kernel(x):
    def body(
        x_hbm,
        out_hbm,
        vbuf_p,
        vbuf_m,
        send_y,
        recv_y,
        send_x,
        recv_x,
        hp_sem,
        hm_sem,
        sy_send,
        sy_recv,
        sx_send,
        sx_recv,
        so_sem,
    ):
        my_x = lax.axis_index("x")
        my_y = lax.axis_index("y")
        peer_x = 1 - my_x
        peer_y = 1 - my_y
        r0 = my_x * HALF
        pr0 = peer_x * HALF
        my_base = my_y * HALF
        peer_base = peer_y * HALF

        hbm_p = []
        hbm_m = []
        for c in range(K):
            col = c * CW
            cp = pltpu.make_async_copy(
                x_hbm.at[0, pl.ds(r0, HALF), pl.ds(peer_base + col, CW)],
                vbuf_p.at[c],
                hp_sem.at[c],
            )
            cp.start()
            hbm_p.append(cp)
        for c in range(K):
            col = c * CW
            cm = pltpu.make_async_copy(
                x_hbm.at[0, pl.ds(r0, HALF), pl.ds(my_base + col, CW)],
                vbuf_m.at[c],
                hm_sem.at[c],
            )
            cm.start()
            hbm_m.append(cm)

        barrier = pltpu.get_barrier_semaphore()
        for dev in ((my_x, peer_y), (peer_x, my_y)):
            pl.semaphore_signal(
                barrier, inc=1, device_id=dev,
                device_id_type=pl.DeviceIdType.MESH,
            )
        pl.semaphore_wait(barrier, 2)

        rdmas_y = []
        for c in range(K):
            hbm_p[c].wait()
            send_y[c] = vbuf_p[c].astype(jnp.bfloat16)
            rdma = pltpu.make_async_remote_copy(
                src_ref=send_y.at[c],
                dst_ref=recv_y.at[c],
                send_sem=sy_send.at[c],
                recv_sem=sy_recv.at[c],
                device_id=(my_x, peer_y),
                device_id_type=pl.DeviceIdType.MESH,
            )
            rdma.start()
            rdmas_y.append(rdma)

        rdmas_x = []
        out_copies = []
        for c in range(K):
            col = c * CW
            rdmas_y[c].wait_recv()
            hbm_m[c].wait()
            send_x[c] = vbuf_m[c].astype(jnp.bfloat16) + recv_y[c]
            rdma = pltpu.make_async_remote_copy(
                src_ref=send_x.at[c],
                dst_ref=recv_x.at[c],
                send_sem=sx_send.at[c],
                recv_sem=sx_recv.at[c],
                device_id=(peer_x, my_y),
                device_id_type=pl.DeviceIdType.MESH,
            )
            rdma.start()
            rdmas_x.append(rdma)

            cp = pltpu.make_async_copy(
                send_x.at[c],
                out_hbm.at[pl.ds(r0, HALF), pl.ds(col, CW)],
                so_sem.at[c],
            )
            cp.start()
            out_copies.append(cp)

        for c in range(K):
            col = c * CW
            rdmas_x[c].wait_recv()
            cp = pltpu.make_async_copy(
                recv_x.at[c],
                out_hbm.at[pl.ds(pr0, HALF), pl.ds(col, CW)],
                so_sem.at[K + c],
            )
            cp.start()
            out_copies.append(cp)

        for cp in out_copies:
            cp.wait()
        for c in range(K):
            rdmas_y[c].wait_send()
            rdmas_x[c].wait_send()

    return pl.pallas_call(
        body,
        out_shape=jax.ShapeDtypeStruct((M, HALF), jnp.bfloat16),
        in_specs=[pl.BlockSpec(memory_space=pltpu.MemorySpace.HBM)],
        out_specs=pl.BlockSpec(memory_space=pltpu.MemorySpace.HBM),
        scratch_shapes=[
            pltpu.VMEM((K, HALF, CW), jnp.float32),
            pltpu.VMEM((K, HALF, CW), jnp.float32),
            pltpu.VMEM((K, HALF, CW), jnp.bfloat16),
            pltpu.VMEM((K, HALF, CW), jnp.bfloat16),
            pltpu.VMEM((K, HALF, CW), jnp.bfloat16),
            pltpu.VMEM((K, HALF, CW), jnp.bfloat16),
            pltpu.SemaphoreType.DMA((K,)),
            pltpu.SemaphoreType.DMA((K,)),
            pltpu.SemaphoreType.DMA((K,)),
            pltpu.SemaphoreType.DMA((K,)),
            pltpu.SemaphoreType.DMA((K,)),
            pltpu.SemaphoreType.DMA((K,)),
            pltpu.SemaphoreType.DMA((2 * K,)),
        ],
        compiler_params=pltpu.CompilerParams(collective_id=0),
    )(x)


# device time: 15182 ns/iter; 1.1135x vs baseline; 1.1128x over previous
import jax
import jax.numpy as jnp
from jax import lax
from jax.experimental import pallas as pl
from jax.experimental.pallas import tpu as pltpu

M = 1024
N = 1024
HALF = 512
K = 4
CW = HALF // K


def kernel(x):
    def body(
        x_hbm,
        out_hbm,
        vbuf_p,
        vbuf_m,
        send_y,
        recv_y,
        send_x,
        recv_x,
        hp_sem,
        hm_sem,
        sy_send,
        sy_recv,
        sx_send,
        sx_recv,
        so_sem,
    ):
        my_x = lax.axis_index("x")
        my_y = lax.axis_index("y")
        peer_x = 1 - my_x
        peer_y = 1 - my_y
        r0 = my_x * HALF
        pr0 = peer_x * HALF
        my_base = my_y * HALF
        peer_base = peer_y * HALF

        hbm_p = []
        hbm_m = []
        for c in range(K):
            col = c * CW
            cp = pltpu.make_async_copy(
                x_hbm.at[0, pl.ds(r0, HALF), pl.ds(peer_base + col, CW)],
                vbuf_p.at[c],
                hp_sem.at[c],
            )
            cp.start()
            hbm_p.append(cp)
        for c in range(K):
            col = c * CW
            cm = pltpu.make_async_copy(
                x_hbm.at[0, pl.ds(r0, HALF), pl.ds(my_base + col, CW)],
                vbuf_m.at[c],
                hm_sem.at[c],
            )
            cm.start()
            hbm_m.append(cm)

        barrier = pltpu.get_barrier_semaphore()
        for dev in ((my_x, peer_y), (peer_x, my_y)):
            pl.semaphore_signal(
                barrier, inc=1, device_id=dev,
                device_id_type=pl.DeviceIdType.MESH,
            )
        pl.semaphore_wait(barrier, 2)

        rdmas_y = []
        for c in range(K):
            hbm_p[c].wait()
            send_y[c] = vbuf_p[c].astype(jnp.bfloat16)
            rdma = pltpu.make_async_remote_copy(
                src_ref=send_y.at[c],
                dst_ref=recv_y.at[c],
                send_sem=sy_send.at[c],
                recv_sem=sy_recv.at[c],
                device_id=(my_x, peer_y),
                device_id_type=pl.DeviceIdType.MESH,
            )
            rdma.start()
            rdmas_y.append(rdma)

        rdmas_x = []
        out_copies = []
        for c in range(K):
            col = c * CW
            rdmas_y[c].wait_recv()
            hbm_m[c].wait()
            send_x[c] = vbuf_m[c].astype(jnp.bfloat16) + recv_y[c]
            rdma = pltpu.make_async_remote_copy(
                src_ref=send_x.at[c],
                dst_ref=recv_x.at[c],
                send_sem=sx_send.at[c],
                recv_sem=sx_recv.at[c],
                device_id=(peer_x, my_y),
                device_id_type=pl.DeviceIdType.MESH,
            )
            rdma.start()
            rdmas_x.append(rdma)

            cp = pltpu.make_async_copy(
                send_x.at[c],
                out_hbm.at[pl.ds(r0, HALF), pl.ds(col, CW)],
                so_sem.at[c],
            )
            cp.start()
            out_copies.append(cp)

        for c in range(K):
            col = c * CW
            rdmas_x[c].wait_recv()
            cp = pltpu.make_async_copy(
                recv_x.at[c],
                out_hbm.at[pl.ds(pr0, HALF), pl.ds(col, CW)],
                so_sem.at[K + c],
            )
            cp.start()
            out_copies.append(cp)

        for cp in out_copies:
            cp.wait()
        for c in range(K):
            rdmas_y[c].wait_send()
            rdmas_x[c].wait_send()

    return pl.pallas_call(
        body,
        out_shape=jax.ShapeDtypeStruct((M, HALF), jnp.bfloat16),
        in_specs=[pl.BlockSpec(memory_space=pltpu.MemorySpace.HBM)],
        out_specs=pl.BlockSpec(memory_space=pltpu.MemorySpace.HBM),
        scratch_shapes=[
            pltpu.VMEM((K, HALF, CW), jnp.float32),
            pltpu.VMEM((K, HALF, CW), jnp.float32),
            pltpu.VMEM((K, HALF, CW), jnp.bfloat16),
            pltpu.VMEM((K, HALF, CW), jnp.bfloat16),
            pltpu.VMEM((K, HALF, CW), jnp.bfloat16),
            pltpu.VMEM((K, HALF, CW), jnp.bfloat16),
            pltpu.SemaphoreType.DMA((K,)),
            pltpu.SemaphoreType.DMA((K,)),
            pltpu.SemaphoreType.DMA((K,)),
            pltpu.SemaphoreType.DMA((K,)),
            pltpu.SemaphoreType.DMA((K,)),
            pltpu.SemaphoreType.DMA((K,)),
            pltpu.SemaphoreType.DMA((2 * K,)),
        ],
        compiler_params=pltpu.CompilerParams(collective_id=0),
    )(pltpu.with_memory_space_constraint(x, pltpu.MemorySpace.HBM))


# device time: 14760 ns/iter; 1.1453x vs baseline; 1.0286x over previous
import jax
import jax.numpy as jnp
from jax import lax
from jax.experimental import pallas as pl
from jax.experimental.pallas import tpu as pltpu

M = 1024
N = 1024
HALF = 512
K = 8
RH = HALF // K


def kernel(x):
    def body(
        x_hbm,
        out_hbm,
        vbuf_p,
        vbuf_m,
        send_y,
        recv_y,
        send_x,
        recv_x,
        hp_sem,
        hm_sem,
        sy_send,
        sy_recv,
        sx_send,
        sx_recv,
        so_sem,
    ):
        my_x = lax.axis_index("x")
        my_y = lax.axis_index("y")
        peer_x = 1 - my_x
        peer_y = 1 - my_y
        r0 = my_x * HALF
        pr0 = peer_x * HALF
        my_base = my_y * HALF
        peer_base = peer_y * HALF

        barrier = pltpu.get_barrier_semaphore()
        for dev in ((my_x, peer_y), (peer_x, my_y)):
            pl.semaphore_signal(
                barrier, inc=1, device_id=dev,
                device_id_type=pl.DeviceIdType.MESH,
            )

        hbm_p = []
        hbm_m = []
        for c in range(K):
            cp = pltpu.make_async_copy(
                x_hbm.at[0, pl.ds(r0 + c * RH, RH), pl.ds(peer_base, HALF)],
                vbuf_p.at[c],
                hp_sem.at[c],
            )
            cp.start()
            hbm_p.append(cp)
        for c in range(K):
            cm = pltpu.make_async_copy(
                x_hbm.at[0, pl.ds(r0 + c * RH, RH), pl.ds(my_base, HALF)],
                vbuf_m.at[c],
                hm_sem.at[c],
            )
            cm.start()
            hbm_m.append(cm)

        pl.semaphore_wait(barrier, 2)

        rdmas_y = []
        for c in range(K):
            hbm_p[c].wait()
            send_y[c] = vbuf_p[c].astype(jnp.bfloat16)
            rdma = pltpu.make_async_remote_copy(
                src_ref=send_y.at[c],
                dst_ref=recv_y.at[c],
                send_sem=sy_send.at[c],
                recv_sem=sy_recv.at[c],
                device_id=(my_x, peer_y),
                device_id_type=pl.DeviceIdType.MESH,
            )
            rdma.start()
            rdmas_y.append(rdma)

        rdmas_x = []
        out_copies = []
        for c in range(K):
            rdmas_y[c].wait_recv()
            hbm_m[c].wait()
            send_x[c] = vbuf_m[c].astype(jnp.bfloat16) + recv_y[c]
            rdma = pltpu.make_async_remote_copy(
                src_ref=send_x.at[c],
                dst_ref=recv_x.at[c],
                send_sem=sx_send.at[c],
                recv_sem=sx_recv.at[c],
                device_id=(peer_x, my_y),
                device_id_type=pl.DeviceIdType.MESH,
            )
            rdma.start()
            rdmas_x.append(rdma)

            cp = pltpu.make_async_copy(
                send_x.at[c],
                out_hbm.at[pl.ds(r0 + c * RH, RH), :],
                so_sem.at[c],
            )
            cp.start()
            out_copies.append(cp)

        for c in range(K):
            rdmas_x[c].wait_recv()
            cp = pltpu.make_async_copy(
                recv_x.at[c],
                out_hbm.at[pl.ds(pr0 + c * RH, RH), :],
                so_sem.at[K + c],
            )
            cp.start()
            out_copies.append(cp)

        for cp in out_copies:
            cp.wait()
        for c in range(K):
            rdmas_y[c].wait_send()
            rdmas_x[c].wait_send()

    return pl.pallas_call(
        body,
        out_shape=jax.ShapeDtypeStruct((M, HALF), jnp.bfloat16),
        in_specs=[pl.BlockSpec(memory_space=pltpu.MemorySpace.HBM)],
        out_specs=pl.BlockSpec(memory_space=pltpu.MemorySpace.HBM),
        scratch_shapes=[
            pltpu.VMEM((K, RH, HALF), jnp.float32),
            pltpu.VMEM((K, RH, HALF), jnp.float32),
            pltpu.VMEM((K, RH, HALF), jnp.bfloat16),
            pltpu.VMEM((K, RH, HALF), jnp.bfloat16),
            pltpu.VMEM((K, RH, HALF), jnp.bfloat16),
            pltpu.VMEM((K, RH, HALF), jnp.bfloat16),
            pltpu.SemaphoreType.DMA((K,)),
            pltpu.SemaphoreType.DMA((K,)),
            pltpu.SemaphoreType.DMA((K,)),
            pltpu.SemaphoreType.DMA((K,)),
            pltpu.SemaphoreType.DMA((K,)),
            pltpu.SemaphoreType.DMA((K,)),
            pltpu.SemaphoreType.DMA((2 * K,)),
        ],
        compiler_params=pltpu.CompilerParams(collective_id=0),
    )(pltpu.with_memory_space_constraint(x, pltpu.MemorySpace.HBM))
